# baseline (speedup 1.0000x reference)
import functools

import jax
import jax.numpy as jnp
from jax import lax
from jax.experimental import pallas as pl
from jax.experimental.pallas import tpu as pltpu
from jax.experimental.pallas import tpu_sc as plsc

D = 128
DEG = 32
B = 10000
L = 16
NVREG = D // L

C = 4
ROWS = C * DEG
NCHUNK = B // C
NW = 32
TBASE = 78
NEXTRA = 2
TMAX = TBASE + 2
NBUF = 4
UNROLL = 8

_mesh = plsc.VectorSubcoreMesh(core_axis_name="c", subcore_axis_name="s")


@functools.partial(
    pl.kernel,
    mesh=_mesh,
    out_type=jax.ShapeDtypeStruct((B, D), jnp.float32),
    scratch_types=[
        pltpu.VMEM((TMAX * ROWS,), jnp.int32),
        pltpu.VMEM((ROWS, D), jnp.float32),
        pltpu.VMEM((ROWS, D), jnp.float32),
        pltpu.VMEM((ROWS, D), jnp.float32),
        pltpu.VMEM((ROWS, D), jnp.float32),
        pltpu.VMEM((2, C, D), jnp.float32),
        pltpu.SemaphoreType.DMA,
        pltpu.SemaphoreType.DMA,
        pltpu.SemaphoreType.DMA,
        pltpu.SemaphoreType.DMA,
        pltpu.SemaphoreType.DMA,
        pltpu.SemaphoreType.DMA,
    ],
)
def _gather_mean(idx_hbm, table_hbm, out_hbm, idx_v, rows0, rows1, rows2,
                 rows3, out_stage, sem0, sem1, sem2, sem3, out_sem0,
                 out_sem1):
    nc = 2
    wid = lax.axis_index("s") * nc + lax.axis_index("c")
    base_chunk = wid * TBASE + 2 * jnp.minimum(wid, NEXTRA)
    n_w = jnp.where(wid < NEXTRA, TBASE + 2, TBASE)
    rows_bufs = (rows0, rows1, rows2, rows3)
    sems = (sem0, sem1, sem2, sem3)
    out_sems = (out_sem0, out_sem1)

    pltpu.sync_copy(idx_hbm.at[pl.ds(base_chunk * ROWS, TBASE * ROWS)],
                    idx_v.at[pl.ds(0, TBASE * ROWS)])

    @pl.when(wid < NEXTRA)
    def _():
        pltpu.sync_copy(
            idx_hbm.at[pl.ds((base_chunk + TBASE) * ROWS, 2 * ROWS)],
            idx_v.at[pl.ds(TBASE * ROWS, 2 * ROWS)])

    for t0 in range(NBUF - 1):
        pltpu.async_copy(
            table_hbm.at[idx_v.at[pl.ds(t0 * ROWS, ROWS)]],
            rows_bufs[t0], sems[t0])

    def accumulate(t, rows_v, stage):
        for n in range(C):
            def row_body(r, accs):
                new = accs
                for u in range(UNROLL):
                    row = n * DEG + r * UNROLL + u
                    new = tuple(
                        new[d] + rows_v[row, pl.ds(d * L, L)]
                        for d in range(NVREG)
                    )
                return new

            accs = lax.fori_loop(
                0, DEG // UNROLL, row_body,
                tuple(jnp.zeros((L,), jnp.float32) for _ in range(NVREG)),
            )
            for d in range(NVREG):
                out_stage[stage, n, pl.ds(d * L, L)] = accs[d] * (1.0 / DEG)

    def outer(i, carry):
        for b in range(NBUF):
            t = i * NBUF + b
            gbuf = b % NBUF
            sbuf = b % 2

            @pl.when(t + NBUF - 1 < n_w)
            def _():
                pltpu.async_copy(
                    table_hbm.at[
                        idx_v.at[pl.ds((t + NBUF - 1) * ROWS, ROWS)]],
                    rows_bufs[(b + NBUF - 1) % NBUF],
                    sems[(b + NBUF - 1) % NBUF])

            @pl.when(t < n_w)
            def _():
                pltpu.make_async_copy(
                    table_hbm.at[idx_v.at[pl.ds(t * ROWS, ROWS)]],
                    rows_bufs[gbuf], sems[gbuf]).wait()

                @pl.when(t >= 2)
                def _():
                    pltpu.make_async_copy(
                        out_stage.at[sbuf],
                        out_hbm.at[pl.ds((base_chunk + t - 2) * C, C)],
                        out_sems[sbuf]).wait()

                accumulate(t, rows_bufs[gbuf], sbuf)
                pltpu.async_copy(
                    out_stage.at[sbuf],
                    out_hbm.at[pl.ds((base_chunk + t) * C, C)],
                    out_sems[sbuf])

        return carry

    lax.fori_loop(0, (TMAX + NBUF - 1) // NBUF, outer, 0)

    for k in range(2):
        pltpu.make_async_copy(
            out_stage.at[k],
            out_hbm.at[pl.ds((base_chunk + n_w - 2 + k) * C, C)],
            out_sems[k]).wait()


def kernel(nodes, to_neighs, v2e_weight):
    del nodes
    idx_flat = to_neighs.reshape(-1)
    return _gather_mean(idx_flat, v2e_weight)


# speedup vs baseline: 1.2405x; 1.2405x over previous
import functools

import jax
import jax.numpy as jnp
from jax import lax
from jax.experimental import pallas as pl
from jax.experimental.pallas import tpu as pltpu
from jax.experimental.pallas import tpu_sc as plsc

D = 128
DEG = 32
B = 10000
L = 16
NVREG = D // L

C = 4
ROWS = C * DEG
NCHUNK = B // C
NW = 32
TBASE = 78
NEXTRA = 2
TMAX = TBASE + 2
NBUF = 4
UNROLL = 2

_mesh = plsc.VectorSubcoreMesh(core_axis_name="c", subcore_axis_name="s")


@functools.partial(
    pl.kernel,
    mesh=_mesh,
    out_type=jax.ShapeDtypeStruct((B, D), jnp.float32),
    scratch_types=[
        pltpu.VMEM((TMAX * ROWS,), jnp.int32),
        pltpu.VMEM((ROWS, D), jnp.float32),
        pltpu.VMEM((ROWS, D), jnp.float32),
        pltpu.VMEM((ROWS, D), jnp.float32),
        pltpu.VMEM((ROWS, D), jnp.float32),
        pltpu.VMEM((2, C, D), jnp.float32),
        pltpu.SemaphoreType.DMA,
        pltpu.SemaphoreType.DMA,
        pltpu.SemaphoreType.DMA,
        pltpu.SemaphoreType.DMA,
        pltpu.SemaphoreType.DMA,
        pltpu.SemaphoreType.DMA,
    ],
)
def _gather_mean(idx_hbm, table_hbm, out_hbm, idx_v, rows0, rows1, rows2,
                 rows3, out_stage, sem0, sem1, sem2, sem3, out_sem0,
                 out_sem1):
    nc = 2
    wid = lax.axis_index("s") * nc + lax.axis_index("c")
    base_chunk = wid * TBASE + 2 * jnp.minimum(wid, NEXTRA)
    n_w = jnp.where(wid < NEXTRA, TBASE + 2, TBASE)
    rows_bufs = (rows0, rows1, rows2, rows3)
    sems = (sem0, sem1, sem2, sem3)
    out_sems = (out_sem0, out_sem1)

    pltpu.sync_copy(idx_hbm.at[pl.ds(base_chunk * ROWS, TBASE * ROWS)],
                    idx_v.at[pl.ds(0, TBASE * ROWS)])

    @pl.when(wid < NEXTRA)
    def _():
        pltpu.sync_copy(
            idx_hbm.at[pl.ds((base_chunk + TBASE) * ROWS, 2 * ROWS)],
            idx_v.at[pl.ds(TBASE * ROWS, 2 * ROWS)])

    for t0 in range(NBUF - 1):
        pltpu.async_copy(
            table_hbm.at[idx_v.at[pl.ds(t0 * ROWS, ROWS)]],
            rows_bufs[t0], sems[t0])

    def accumulate(t, rows_v, stage):
        for n in range(C):
            def row_body(r, accs):
                new = accs
                for u in range(UNROLL):
                    row = n * DEG + r * UNROLL + u
                    new = tuple(
                        new[d] + rows_v[row, pl.ds(d * L, L)]
                        for d in range(NVREG)
                    )
                return new

            accs = lax.fori_loop(
                0, DEG // UNROLL, row_body,
                tuple(jnp.zeros((L,), jnp.float32) for _ in range(NVREG)),
            )
            for d in range(NVREG):
                out_stage[stage, n, pl.ds(d * L, L)] = accs[d] * (1.0 / DEG)

    def outer(i, carry):
        for b in range(NBUF):
            t = i * NBUF + b
            gbuf = b % NBUF
            sbuf = b % 2

            @pl.when(t + NBUF - 1 < n_w)
            def _():
                pltpu.async_copy(
                    table_hbm.at[
                        idx_v.at[pl.ds((t + NBUF - 1) * ROWS, ROWS)]],
                    rows_bufs[(b + NBUF - 1) % NBUF],
                    sems[(b + NBUF - 1) % NBUF])

            @pl.when(t < n_w)
            def _():
                pltpu.make_async_copy(
                    table_hbm.at[idx_v.at[pl.ds(t * ROWS, ROWS)]],
                    rows_bufs[gbuf], sems[gbuf]).wait()

                @pl.when(t >= 2)
                def _():
                    pltpu.make_async_copy(
                        out_stage.at[sbuf],
                        out_hbm.at[pl.ds((base_chunk + t - 2) * C, C)],
                        out_sems[sbuf]).wait()

                accumulate(t, rows_bufs[gbuf], sbuf)
                pltpu.async_copy(
                    out_stage.at[sbuf],
                    out_hbm.at[pl.ds((base_chunk + t) * C, C)],
                    out_sems[sbuf])

        return carry

    lax.fori_loop(0, (TMAX + NBUF - 1) // NBUF, outer, 0)

    for k in range(2):
        pltpu.make_async_copy(
            out_stage.at[k],
            out_hbm.at[pl.ds((base_chunk + n_w - 2 + k) * C, C)],
            out_sems[k]).wait()


def kernel(nodes, to_neighs, v2e_weight):
    del nodes
    idx_flat = to_neighs.reshape(-1)
    return _gather_mean(idx_flat, v2e_weight)


# speedup vs baseline: 1.2454x; 1.0040x over previous
import functools

import jax
import jax.numpy as jnp
from jax import lax
from jax.experimental import pallas as pl
from jax.experimental.pallas import tpu as pltpu
from jax.experimental.pallas import tpu_sc as plsc

D = 128
DEG = 32
B = 10000
L = 16
NVREG = D // L

C = 4
ROWS = C * DEG
NCHUNK = B // C
NW = 32
TBASE = 78
NEXTRA = 2
TMAX = TBASE + 2
NBUF = 4
UNROLL = 1

_mesh = plsc.VectorSubcoreMesh(core_axis_name="c", subcore_axis_name="s")


@functools.partial(
    pl.kernel,
    mesh=_mesh,
    out_type=jax.ShapeDtypeStruct((B, D), jnp.float32),
    scratch_types=[
        pltpu.VMEM((TMAX * ROWS,), jnp.int32),
        pltpu.VMEM((ROWS, D), jnp.float32),
        pltpu.VMEM((ROWS, D), jnp.float32),
        pltpu.VMEM((ROWS, D), jnp.float32),
        pltpu.VMEM((ROWS, D), jnp.float32),
        pltpu.VMEM((2, C, D), jnp.float32),
        pltpu.SemaphoreType.DMA,
        pltpu.SemaphoreType.DMA,
        pltpu.SemaphoreType.DMA,
        pltpu.SemaphoreType.DMA,
        pltpu.SemaphoreType.DMA,
        pltpu.SemaphoreType.DMA,
    ],
)
def _gather_mean(idx_hbm, table_hbm, out_hbm, idx_v, rows0, rows1, rows2,
                 rows3, out_stage, sem0, sem1, sem2, sem3, out_sem0,
                 out_sem1):
    nc = 2
    wid = lax.axis_index("s") * nc + lax.axis_index("c")
    base_chunk = wid * TBASE + 2 * jnp.minimum(wid, NEXTRA)
    n_w = jnp.where(wid < NEXTRA, TBASE + 2, TBASE)
    rows_bufs = (rows0, rows1, rows2, rows3)
    sems = (sem0, sem1, sem2, sem3)
    out_sems = (out_sem0, out_sem1)

    pltpu.sync_copy(idx_hbm.at[pl.ds(base_chunk * ROWS, TBASE * ROWS)],
                    idx_v.at[pl.ds(0, TBASE * ROWS)])

    @pl.when(wid < NEXTRA)
    def _():
        pltpu.sync_copy(
            idx_hbm.at[pl.ds((base_chunk + TBASE) * ROWS, 2 * ROWS)],
            idx_v.at[pl.ds(TBASE * ROWS, 2 * ROWS)])

    for t0 in range(NBUF - 1):
        pltpu.async_copy(
            table_hbm.at[idx_v.at[pl.ds(t0 * ROWS, ROWS)]],
            rows_bufs[t0], sems[t0])

    def accumulate(t, rows_v, stage):
        for n in range(C):
            def row_body(r, accs):
                new = accs
                for u in range(UNROLL):
                    row = n * DEG + r * UNROLL + u
                    new = tuple(
                        new[d] + rows_v[row, pl.ds(d * L, L)]
                        for d in range(NVREG)
                    )
                return new

            accs = lax.fori_loop(
                0, DEG // UNROLL, row_body,
                tuple(jnp.zeros((L,), jnp.float32) for _ in range(NVREG)),
            )
            for d in range(NVREG):
                out_stage[stage, n, pl.ds(d * L, L)] = accs[d] * (1.0 / DEG)

    def outer(i, carry):
        for b in range(NBUF):
            t = i * NBUF + b
            gbuf = b % NBUF
            sbuf = b % 2

            @pl.when(t + NBUF - 1 < n_w)
            def _():
                pltpu.async_copy(
                    table_hbm.at[
                        idx_v.at[pl.ds((t + NBUF - 1) * ROWS, ROWS)]],
                    rows_bufs[(b + NBUF - 1) % NBUF],
                    sems[(b + NBUF - 1) % NBUF])

            @pl.when(t < n_w)
            def _():
                pltpu.make_async_copy(
                    table_hbm.at[idx_v.at[pl.ds(t * ROWS, ROWS)]],
                    rows_bufs[gbuf], sems[gbuf]).wait()

                @pl.when(t >= 2)
                def _():
                    pltpu.make_async_copy(
                        out_stage.at[sbuf],
                        out_hbm.at[pl.ds((base_chunk + t - 2) * C, C)],
                        out_sems[sbuf]).wait()

                accumulate(t, rows_bufs[gbuf], sbuf)
                pltpu.async_copy(
                    out_stage.at[sbuf],
                    out_hbm.at[pl.ds((base_chunk + t) * C, C)],
                    out_sems[sbuf])

        return carry

    lax.fori_loop(0, (TMAX + NBUF - 1) // NBUF, outer, 0)

    for k in range(2):
        pltpu.make_async_copy(
            out_stage.at[k],
            out_hbm.at[pl.ds((base_chunk + n_w - 2 + k) * C, C)],
            out_sems[k]).wait()


def kernel(nodes, to_neighs, v2e_weight):
    del nodes
    idx_flat = to_neighs.reshape(-1)
    return _gather_mean(idx_flat, v2e_weight)


# speedup vs baseline: 1.2697x; 1.0195x over previous
import functools

import jax
import jax.numpy as jnp
from jax import lax
from jax.experimental import pallas as pl
from jax.experimental.pallas import tpu as pltpu
from jax.experimental.pallas import tpu_sc as plsc

D = 128
DEG = 32
B = 10000
L = 16
NVREG = D // L

C = 4
ROWS = C * DEG
NCHUNK = B // C
NW = 32
TBASE = 78
NEXTRA = 2
TMAX = TBASE + 2
NBUF = 4
UNROLL = 1

_mesh = plsc.VectorSubcoreMesh(core_axis_name="c", subcore_axis_name="s")


@functools.partial(
    pl.kernel,
    mesh=_mesh,
    out_type=jax.ShapeDtypeStruct((B, D), jnp.float32),
    scratch_types=[
        pltpu.VMEM((TMAX * ROWS,), jnp.int32),
        pltpu.VMEM((ROWS, D), jnp.float32),
        pltpu.VMEM((ROWS, D), jnp.float32),
        pltpu.VMEM((ROWS, D), jnp.float32),
        pltpu.VMEM((ROWS, D), jnp.float32),
        pltpu.VMEM((2, C, D), jnp.float32),
        pltpu.SemaphoreType.DMA,
        pltpu.SemaphoreType.DMA,
        pltpu.SemaphoreType.DMA,
        pltpu.SemaphoreType.DMA,
        pltpu.SemaphoreType.DMA,
        pltpu.SemaphoreType.DMA,
    ],
)
def _gather_mean(idx_hbm, table_hbm, out_hbm, idx_v, rows0, rows1, rows2,
                 rows3, out_stage, sem0, sem1, sem2, sem3, out_sem0,
                 out_sem1):
    nc = 2
    wid = lax.axis_index("s") * nc + lax.axis_index("c")
    base_chunk = wid * TBASE + 2 * jnp.minimum(wid, NEXTRA)
    n_w = jnp.where(wid < NEXTRA, TBASE + 2, TBASE)
    rows_bufs = (rows0, rows1, rows2, rows3)
    sems = (sem0, sem1, sem2, sem3)
    out_sems = (out_sem0, out_sem1)

    head = (NBUF - 1) * ROWS
    pltpu.sync_copy(idx_hbm.at[pl.ds(base_chunk * ROWS, head)],
                    idx_v.at[pl.ds(0, head)])
    for t0 in range(NBUF - 1):
        pltpu.async_copy(
            table_hbm.at[idx_v.at[pl.ds(t0 * ROWS, ROWS)]],
            rows_bufs[t0], sems[t0])

    pltpu.sync_copy(
        idx_hbm.at[pl.ds(base_chunk * ROWS + head, TBASE * ROWS - head)],
        idx_v.at[pl.ds(head, TBASE * ROWS - head)])

    @pl.when(wid < NEXTRA)
    def _():
        pltpu.sync_copy(
            idx_hbm.at[pl.ds((base_chunk + TBASE) * ROWS, 2 * ROWS)],
            idx_v.at[pl.ds(TBASE * ROWS, 2 * ROWS)])

    def accumulate(t, rows_v, stage):
        for n in range(C):
            def row_body(r, accs):
                new = accs
                for u in range(UNROLL):
                    row = n * DEG + r * UNROLL + u
                    new = tuple(
                        new[d] + rows_v[row, pl.ds(d * L, L)]
                        for d in range(NVREG)
                    )
                return new

            accs = lax.fori_loop(
                0, DEG // UNROLL, row_body,
                tuple(jnp.zeros((L,), jnp.float32) for _ in range(NVREG)),
            )
            for d in range(NVREG):
                out_stage[stage, n, pl.ds(d * L, L)] = accs[d] * (1.0 / DEG)

    def outer(i, carry):
        for b in range(NBUF):
            t = i * NBUF + b
            gbuf = b % NBUF
            sbuf = b % 2

            @pl.when(t + NBUF - 1 < n_w)
            def _():
                pltpu.async_copy(
                    table_hbm.at[
                        idx_v.at[pl.ds((t + NBUF - 1) * ROWS, ROWS)]],
                    rows_bufs[(b + NBUF - 1) % NBUF],
                    sems[(b + NBUF - 1) % NBUF])

            @pl.when(t < n_w)
            def _():
                pltpu.make_async_copy(
                    table_hbm.at[idx_v.at[pl.ds(t * ROWS, ROWS)]],
                    rows_bufs[gbuf], sems[gbuf]).wait()

                @pl.when(t >= 2)
                def _():
                    pltpu.make_async_copy(
                        out_stage.at[sbuf],
                        out_hbm.at[pl.ds((base_chunk + t - 2) * C, C)],
                        out_sems[sbuf]).wait()

                accumulate(t, rows_bufs[gbuf], sbuf)
                pltpu.async_copy(
                    out_stage.at[sbuf],
                    out_hbm.at[pl.ds((base_chunk + t) * C, C)],
                    out_sems[sbuf])

        return carry

    lax.fori_loop(0, (TMAX + NBUF - 1) // NBUF, outer, 0)

    for k in range(2):
        pltpu.make_async_copy(
            out_stage.at[k],
            out_hbm.at[pl.ds((base_chunk + n_w - 2 + k) * C, C)],
            out_sems[k]).wait()


def kernel(nodes, to_neighs, v2e_weight):
    del nodes
    idx_flat = to_neighs.reshape(-1)
    return _gather_mean(idx_flat, v2e_weight)
